# Initial kernel scaffold; baseline (speedup 1.0000x reference)
import jax
import jax.numpy as jnp
from jax.experimental import pallas as pl


def kernel(x, edge_index, edge_attr, W, att_src, att_dst, W_edge, att_edge, bias, gamma, beta):
    raise NotImplementedError("write your pallas kernel here")


# speedup vs baseline: 11.5530x; 11.5530x over previous
import functools

import jax
import jax.numpy as jnp
from jax import lax
from jax.experimental import pallas as pl
from jax.experimental.pallas import tpu as pltpu
from jax.experimental.pallas import tpu_sc as plsc

N = 10000
E = 160000
IN = 256
H = 4
C = 128
HC = H * C
ED = 16

RB = 400
EB = 2000
NS = 16
EPT = E // NS
CH = 80
NCHUNK = EPT // CH
RPT = 624


def _proj_body(x_ref, w_ref, as_ref, ad_ref, xp_ref, xph_ref, a4s_ref, a4d_ref):
    h = pl.program_id(1)
    xpb = jnp.dot(x_ref[...], w_ref[...], preferred_element_type=jnp.float32)
    xp_ref[...] = xpb
    xph_ref[...] = xpb
    a_s = jnp.dot(xpb, as_ref[...], preferred_element_type=jnp.float32)
    a_d = jnp.dot(xpb, ad_ref[...], preferred_element_type=jnp.float32)
    col = lax.broadcasted_iota(jnp.int32, (1, H), 1)

    @pl.when(h == 0)
    def _():
        a4s_ref[...] = jnp.zeros_like(a4s_ref)
        a4d_ref[...] = jnp.zeros_like(a4d_ref)

    a4s_ref[...] = jnp.where(col == h, a_s, a4s_ref[...])
    a4d_ref[...] = jnp.where(col == h, a_d, a4d_ref[...])


_proj = pl.pallas_call(
    _proj_body,
    grid=(N // RB, H),
    in_specs=[
        pl.BlockSpec((RB, IN), lambda i, h: (i, 0)),
        pl.BlockSpec((IN, C), lambda i, h: (0, h)),
        pl.BlockSpec((C, H), lambda i, h: (0, 0)),
        pl.BlockSpec((C, H), lambda i, h: (0, 0)),
    ],
    out_specs=[
        pl.BlockSpec((RB, C), lambda i, h: (i, h)),
        pl.BlockSpec((RB, C), lambda i, h: (h * (N // RB) + i, 0)),
        pl.BlockSpec((RB, H), lambda i, h: (i, 0)),
        pl.BlockSpec((RB, H), lambda i, h: (i, 0)),
    ],
    out_shape=[
        jax.ShapeDtypeStruct((N, HC), jnp.float32),
        jax.ShapeDtypeStruct((H * N, C), jnp.float32),
        jax.ShapeDtypeStruct((N, H), jnp.float32),
        jax.ShapeDtypeStruct((N, H), jnp.float32),
    ],
)


def _edgeproj_body(ea_ref, we_ref, ae_ref, z0_ref):
    ve = jnp.sum(we_ref[...].reshape(ED, H, C) * ae_ref[...][None], axis=2)
    z0_ref[...] = jnp.dot(ea_ref[...], ve, preferred_element_type=jnp.float32)


_edgeproj = pl.pallas_call(
    _edgeproj_body,
    grid=(E // EB,),
    in_specs=[
        pl.BlockSpec((EB, ED), lambda i: (i, 0)),
        pl.BlockSpec((ED, HC), lambda i: (0, 0)),
        pl.BlockSpec((H, C), lambda i: (0, 0)),
    ],
    out_specs=pl.BlockSpec((EB, H), lambda i: (i, 0)),
    out_shape=jax.ShapeDtypeStruct((E, H), jnp.float32),
)


def _edge_pass_body(xph, srcv, dstv, z0, asrc4, adst4, xagg, den_out, cz_out,
                    accm, accd, accc, az0, az1, az2, az3,
                    asrc_t, adst_t, srcb, dstb, gidx, zb,
                    rows, exb, oneb, zcb, zrb, fb, sem):
    c = lax.axis_index("c")
    s = lax.axis_index("s")
    r0 = s * RPT
    TAIL = N - NS * RPT
    iota16 = jnp.arange(16, dtype=jnp.int32)
    zeros16f = jnp.zeros((16,), jnp.float32)

    for g in range(CH // 16):
        oneb[0, pl.ds(16 * g, 16)] = zeros16f + 1.0
    for q in range(RPT // 16):
        zrb[0, pl.ds(16 * q, 16)] = zeros16f

    def head_pass(p, carry):
        h = 2 * c + p
        hN = h * N
        czon = jnp.logical_and(c == 0, p == 0)
        pltpu.sync_copy(asrc4.at[pl.ds(hN, N)], asrc_t)
        pltpu.sync_copy(adst4.at[pl.ds(hN, N)], adst_t)
        for i in range(CH):
            for q in range(C // 16):
                rows[i, pl.ds(16 * q, 16)] = zeros16f
        for k in range(7):
            pltpu.sync_copy(rows, accm.at[pl.ds(r0 + k * CH, CH)])
        pltpu.sync_copy(rows.at[pl.ds(0, RPT - 7 * CH)],
                        accm.at[pl.ds(r0 + 7 * CH, RPT - 7 * CH)])
        for acc in (accd, accc, az0, az1, az2, az3):
            pltpu.sync_copy(zrb.at[0], acc.at[pl.ds(r0, RPT)])

        @pl.when(s == NS - 1)
        def _():
            pltpu.sync_copy(rows.at[pl.ds(0, TAIL)],
                            accm.at[pl.ds(NS * RPT, TAIL)])
            for acc in (accd, accc, az0, az1, az2, az3):
                pltpu.sync_copy(zrb.at[0, pl.ds(0, TAIL)],
                                acc.at[pl.ds(NS * RPT, TAIL)])

        plsc.subcore_barrier()
        hsplat = jnp.full((16,), h, jnp.int32)

        def chunk(j, carry2):
            e0 = s * EPT + j * CH
            pltpu.sync_copy(srcv.at[pl.ds(e0, CH)], srcb.at[0])
            pltpu.sync_copy(dstv.at[pl.ds(e0, CH)], dstb.at[0])
            pltpu.sync_copy(z0.at[pl.ds(e0 * H, CH * H)], zb)
            for g in range(CH // 16):
                li = iota16 + 16 * g
                sv = srcb[0, pl.ds(16 * g, 16)]
                dv = dstb[0, pl.ds(16 * g, 16)]
                gidx[0, pl.ds(16 * g, 16)] = sv + hN
                av = plsc.load_gather(asrc_t, [sv])
                bv = plsc.load_gather(adst_t, [dv])
                zv = plsc.load_gather(zb, [li * H + hsplat])
                sa = av + bv + zv
                alpha = jnp.where(sa >= 0.0, sa, 0.2 * sa)
                exb[pl.ds(16 * g, 16)] = jnp.exp(alpha)

            @pl.when(czon)
            def _():
                for g in range(CH // 16):
                    li = iota16 + 16 * g
                    for hc in range(H):
                        zcb[hc, pl.ds(16 * g, 16)] = plsc.load_gather(
                            zb, [li * H + hc])

            pltpu.async_copy(xph.at[gidx.at[0]], rows, sem).wait()

            def scale_row(r, carry3):
                exs = plsc.load_gather(exb, [jnp.zeros((16,), jnp.int32) + r])
                for q in range(C // 16):
                    rows[r, pl.ds(16 * q, 16)] = rows[r, pl.ds(16 * q, 16)] * exs
                return carry3

            lax.fori_loop(0, CH, scale_row, 0)
            didx = dstb.at[0]
            pltpu.sync_copy(rows, accm.at[didx], add=True)
            pltpu.sync_copy(exb, accd.at[didx], add=True)

            @pl.when(czon)
            def _():
                pltpu.sync_copy(oneb.at[0], accc.at[didx], add=True)
                pltpu.sync_copy(zcb.at[0], az0.at[didx], add=True)
                pltpu.sync_copy(zcb.at[1], az1.at[didx], add=True)
                pltpu.sync_copy(zcb.at[2], az2.at[didx], add=True)
                pltpu.sync_copy(zcb.at[3], az3.at[didx], add=True)

            return carry2

        lax.fori_loop(0, NCHUNK, chunk, 0)
        plsc.subcore_barrier()
        pltpu.sync_copy(accm.at[pl.ds(r0, RPT)], xagg.at[pl.ds(hN + r0, RPT)])
        pltpu.sync_copy(accd.at[pl.ds(r0, RPT)], fb.at[0])
        pltpu.sync_copy(fb.at[0], den_out.at[pl.ds(hN + r0, RPT)])

        @pl.when(s == NS - 1)
        def _():
            pltpu.sync_copy(accm.at[pl.ds(NS * RPT, TAIL)],
                            xagg.at[pl.ds(hN + NS * RPT, TAIL)])
            pltpu.sync_copy(accd.at[pl.ds(NS * RPT, TAIL)],
                            fb.at[0, pl.ds(0, TAIL)])
            pltpu.sync_copy(fb.at[0, pl.ds(0, TAIL)],
                            den_out.at[pl.ds(hN + NS * RPT, TAIL)])

        @pl.when(czon)
        def _():
            for q, acc in enumerate((accc, az0, az1, az2, az3)):
                pltpu.sync_copy(acc.at[pl.ds(r0, RPT)], fb.at[0])
                pltpu.sync_copy(fb.at[0], cz_out.at[pl.ds(q * N + r0, RPT)])

            @pl.when(s == NS - 1)
            def _():
                for q, acc in enumerate((accc, az0, az1, az2, az3)):
                    pltpu.sync_copy(acc.at[pl.ds(NS * RPT, TAIL)],
                                    fb.at[0, pl.ds(0, TAIL)])
                    pltpu.sync_copy(fb.at[0, pl.ds(0, TAIL)],
                                    cz_out.at[pl.ds(q * N + NS * RPT, TAIL)])

        plsc.subcore_barrier()
        return carry

    lax.fori_loop(0, 2, head_pass, 0)


@functools.cache
def _build_edge_pass():
    return functools.partial(
        pl.kernel,
        mesh=plsc.VectorSubcoreMesh(core_axis_name="c", subcore_axis_name="s"),
        compiler_params=pltpu.CompilerParams(needs_layout_passes=False),
        out_type=(
            jax.ShapeDtypeStruct((H * N, C), jnp.float32),
            jax.ShapeDtypeStruct((H * N,), jnp.float32),
            jax.ShapeDtypeStruct((5 * N,), jnp.float32),
        ),
        scratch_types=[
            pltpu.VMEM_SHARED((N, C), jnp.float32),
            pltpu.VMEM_SHARED((N,), jnp.float32),
            pltpu.VMEM_SHARED((N,), jnp.float32),
            pltpu.VMEM_SHARED((N,), jnp.float32),
            pltpu.VMEM_SHARED((N,), jnp.float32),
            pltpu.VMEM_SHARED((N,), jnp.float32),
            pltpu.VMEM_SHARED((N,), jnp.float32),
            pltpu.VMEM((N,), jnp.float32),
            pltpu.VMEM((N,), jnp.float32),
            pltpu.VMEM((1, CH), jnp.int32),
            pltpu.VMEM((1, CH), jnp.int32),
            pltpu.VMEM((1, CH), jnp.int32),
            pltpu.VMEM((CH * H,), jnp.float32),
            pltpu.VMEM((CH, C), jnp.float32),
            pltpu.VMEM((CH,), jnp.float32),
            pltpu.VMEM((1, CH), jnp.float32),
            pltpu.VMEM((H, CH), jnp.float32),
            pltpu.VMEM((1, RPT), jnp.float32),
            pltpu.VMEM((1, RPT), jnp.float32),
            pltpu.SemaphoreType.DMA,
        ],
    )(_edge_pass_body)


def _combine_body(xp_ref, xg0, xg1, xg2, xg3, den_ref, cz_ref, as_ref, ad_ref,
                  bias_ref, out_ref, stats_ref):
    i = pl.program_id(0)
    denom4 = den_ref[...]
    cnt = cz_ref[...][:, 0:1]
    zsum4 = cz_ref[...][:, 1:1 + H]
    zloop4 = zsum4 / jnp.maximum(cnt, 1.0)
    sa = as_ref[...] + ad_ref[...] + zloop4
    alpha = jnp.where(sa >= 0.0, sa, 0.2 * sa)
    exloop4 = jnp.exp(alpha)
    jj = lax.broadcasted_iota(jnp.int32, (H, HC), 1) // C
    hh = lax.broadcasted_iota(jnp.int32, (H, HC), 0)
    expand = (jj == hh).astype(jnp.float32)
    xagg = jnp.concatenate([xg0[...], xg1[...], xg2[...], xg3[...]], axis=1)
    num = xagg + xp_ref[...] * jnp.dot(exloop4, expand,
                                       preferred_element_type=jnp.float32)
    den = jnp.dot(denom4 + exloop4, expand,
                  preferred_element_type=jnp.float32) + 1e-16
    out = num / den + bias_ref[...]
    out_ref[...] = out

    @pl.when(i == 0)
    def _():
        stats_ref[...] = jnp.zeros_like(stats_ref)

    sums = jnp.sum(out, axis=0, keepdims=True)
    sq = jnp.sum(out * out, axis=0, keepdims=True)
    stats_ref[...] += jnp.concatenate([sums, sq], axis=0)


_combine = pl.pallas_call(
    _combine_body,
    grid=(N // RB,),
    in_specs=[
        pl.BlockSpec((RB, HC), lambda i: (i, 0)),
        pl.BlockSpec((RB, C), lambda i: (0 * (N // RB) + i, 0)),
        pl.BlockSpec((RB, C), lambda i: (1 * (N // RB) + i, 0)),
        pl.BlockSpec((RB, C), lambda i: (2 * (N // RB) + i, 0)),
        pl.BlockSpec((RB, C), lambda i: (3 * (N // RB) + i, 0)),
        pl.BlockSpec((RB, H), lambda i: (i, 0)),
        pl.BlockSpec((RB, 5), lambda i: (i, 0)),
        pl.BlockSpec((RB, H), lambda i: (i, 0)),
        pl.BlockSpec((RB, H), lambda i: (i, 0)),
        pl.BlockSpec((1, HC), lambda i: (0, 0)),
    ],
    out_specs=[
        pl.BlockSpec((RB, HC), lambda i: (i, 0)),
        pl.BlockSpec((2, HC), lambda i: (0, 0)),
    ],
    out_shape=[
        jax.ShapeDtypeStruct((N, HC), jnp.float32),
        jax.ShapeDtypeStruct((2, HC), jnp.float32),
    ],
)


def _bn_body(o_ref, stats_ref, g_ref, b_ref, out_ref):
    mu = stats_ref[0:1, :] / N
    ex2 = stats_ref[1:2, :] / N
    var = ex2 - mu * mu
    y = (o_ref[...] - mu) * lax.rsqrt(var + 1e-5) * g_ref[...] + b_ref[...]
    out_ref[...] = jnp.where(y > 0.0, y, jnp.exp(y) - 1.0)


_bn_elu = pl.pallas_call(
    _bn_body,
    grid=(N // RB,),
    in_specs=[
        pl.BlockSpec((RB, HC), lambda i: (i, 0)),
        pl.BlockSpec((2, HC), lambda i: (0, 0)),
        pl.BlockSpec((1, HC), lambda i: (0, 0)),
        pl.BlockSpec((1, HC), lambda i: (0, 0)),
    ],
    out_specs=pl.BlockSpec((RB, HC), lambda i: (i, 0)),
    out_shape=jax.ShapeDtypeStruct((N, HC), jnp.float32),
)


def kernel(x, edge_index, edge_attr, W, att_src, att_dst, W_edge, att_edge,
           bias, gamma, beta):
    src = edge_index[0]
    dst = edge_index[1]
    xp, xph, asrc4, adst4 = _proj(x, W, att_src.T, att_dst.T)
    z0 = _edgeproj(edge_attr, W_edge, att_edge)
    xagg, den, cz = _build_edge_pass()(xph, src, dst, z0.reshape(-1),
                                       asrc4.T.reshape(-1), adst4.T.reshape(-1))
    out_pre, stats = _combine(xp, xagg, xagg, xagg, xagg,
                              den.reshape(H, N).T, cz.reshape(5, N).T,
                              asrc4, adst4, bias.reshape(1, HC))
    return _bn_elu(out_pre, stats, gamma.reshape(1, HC), beta.reshape(1, HC))


# speedup vs baseline: 11.7960x; 1.0210x over previous
import functools

import jax
import jax.numpy as jnp
from jax import lax
from jax.experimental import pallas as pl
from jax.experimental.pallas import tpu as pltpu
from jax.experimental.pallas import tpu_sc as plsc

N = 10000
E = 160000
IN = 256
H = 4
C = 128
HC = H * C
ED = 16

RB = 400
EB = 2000
NS = 16
EPT = E // NS
CH = 80
NCHUNK = EPT // CH
RPT = 624


def _proj_body(x_ref, w_ref, as_ref, ad_ref, xp_ref, xph_ref, a4s_ref, a4d_ref):
    h = pl.program_id(1)
    xpb = jnp.dot(x_ref[...], w_ref[...], preferred_element_type=jnp.float32)
    xp_ref[...] = xpb
    xph_ref[...] = xpb
    a_s = jnp.dot(xpb, as_ref[...], preferred_element_type=jnp.float32)
    a_d = jnp.dot(xpb, ad_ref[...], preferred_element_type=jnp.float32)
    col = lax.broadcasted_iota(jnp.int32, (1, H), 1)

    @pl.when(h == 0)
    def _():
        a4s_ref[...] = jnp.zeros_like(a4s_ref)
        a4d_ref[...] = jnp.zeros_like(a4d_ref)

    a4s_ref[...] = jnp.where(col == h, a_s, a4s_ref[...])
    a4d_ref[...] = jnp.where(col == h, a_d, a4d_ref[...])


_proj = pl.pallas_call(
    _proj_body,
    grid=(N // RB, H),
    in_specs=[
        pl.BlockSpec((RB, IN), lambda i, h: (i, 0)),
        pl.BlockSpec((IN, C), lambda i, h: (0, h)),
        pl.BlockSpec((C, H), lambda i, h: (0, 0)),
        pl.BlockSpec((C, H), lambda i, h: (0, 0)),
    ],
    out_specs=[
        pl.BlockSpec((RB, C), lambda i, h: (i, h)),
        pl.BlockSpec((RB, C), lambda i, h: (h * (N // RB) + i, 0)),
        pl.BlockSpec((RB, H), lambda i, h: (i, 0)),
        pl.BlockSpec((RB, H), lambda i, h: (i, 0)),
    ],
    out_shape=[
        jax.ShapeDtypeStruct((N, HC), jnp.float32),
        jax.ShapeDtypeStruct((H * N, C), jnp.float32),
        jax.ShapeDtypeStruct((N, H), jnp.float32),
        jax.ShapeDtypeStruct((N, H), jnp.float32),
    ],
)


def _edgeproj_body(ea_ref, we_ref, ae_ref, z0_ref):
    ve = jnp.sum(we_ref[...].reshape(ED, H, C) * ae_ref[...][None], axis=2)
    z0_ref[...] = jnp.dot(ea_ref[...], ve, preferred_element_type=jnp.float32)


_edgeproj = pl.pallas_call(
    _edgeproj_body,
    grid=(E // EB,),
    in_specs=[
        pl.BlockSpec((EB, ED), lambda i: (i, 0)),
        pl.BlockSpec((ED, HC), lambda i: (0, 0)),
        pl.BlockSpec((H, C), lambda i: (0, 0)),
    ],
    out_specs=pl.BlockSpec((EB, H), lambda i: (i, 0)),
    out_shape=jax.ShapeDtypeStruct((E, H), jnp.float32),
)


def _edge_pass_body(xph, srcv, dstv, z0, asrc4, adst4, xagg, den_out, cz_out,
                    accm, accd, accc, az0, az1, az2, az3,
                    asrc_t, adst_t, srcb, dstb, gidx, zb,
                    rows, exb, oneb, zcb, zrb, fb,
                    sg0, sg1, sg2, sg3):
    c = lax.axis_index("c")
    s = lax.axis_index("s")
    r0 = s * RPT
    TAIL = N - NS * RPT
    iota16 = jnp.arange(16, dtype=jnp.int32)
    zeros16f = jnp.zeros((16,), jnp.float32)

    for g in range(CH // 16):
        oneb[0, pl.ds(16 * g, 16)] = zeros16f + 1.0
    for q in range(RPT // 16):
        zrb[0, pl.ds(16 * q, 16)] = zeros16f

    def head_pass(p, carry):
        h = 2 * c + p
        hN = h * N
        czon = jnp.logical_and(c == 0, p == 0)
        pltpu.sync_copy(asrc4.at[pl.ds(hN, N)], asrc_t)
        pltpu.sync_copy(adst4.at[pl.ds(hN, N)], adst_t)
        for i in range(CH):
            for q in range(C // 16):
                rows[i, pl.ds(16 * q, 16)] = zeros16f
        for k in range(7):
            pltpu.sync_copy(rows, accm.at[pl.ds(r0 + k * CH, CH)])
        pltpu.sync_copy(rows.at[pl.ds(0, RPT - 7 * CH)],
                        accm.at[pl.ds(r0 + 7 * CH, RPT - 7 * CH)])
        for acc in (accd, accc, az0, az1, az2, az3):
            pltpu.sync_copy(zrb.at[0], acc.at[pl.ds(r0, RPT)])

        @pl.when(s == NS - 1)
        def _():
            pltpu.sync_copy(rows.at[pl.ds(0, TAIL)],
                            accm.at[pl.ds(NS * RPT, TAIL)])
            for acc in (accd, accc, az0, az1, az2, az3):
                pltpu.sync_copy(zrb.at[0, pl.ds(0, TAIL)],
                                acc.at[pl.ds(NS * RPT, TAIL)])

        plsc.subcore_barrier()
        hsplat = jnp.full((16,), h, jnp.int32)

        def chunk(j, carry2):
            e0 = s * EPT + j * CH
            pltpu.sync_copy(srcv.at[pl.ds(e0, CH)], srcb.at[0])
            pltpu.sync_copy(dstv.at[pl.ds(e0, CH)], dstb.at[0])
            pltpu.sync_copy(z0.at[pl.ds(e0 * H, CH * H)], zb)
            for g in range(CH // 16):
                li = iota16 + 16 * g
                sv = srcb[0, pl.ds(16 * g, 16)]
                dv = dstb[0, pl.ds(16 * g, 16)]
                gidx[0, pl.ds(16 * g, 16)] = sv + hN
                av = plsc.load_gather(asrc_t, [sv])
                bv = plsc.load_gather(adst_t, [dv])
                zv = plsc.load_gather(zb, [li * H + hsplat])
                sa = av + bv + zv
                alpha = jnp.where(sa >= 0.0, sa, 0.2 * sa)
                exb[pl.ds(16 * g, 16)] = jnp.exp(alpha)

            @pl.when(czon)
            def _():
                for g in range(CH // 16):
                    li = iota16 + 16 * g
                    for hc in range(H):
                        zcb[hc, pl.ds(16 * g, 16)] = plsc.load_gather(
                            zb, [li * H + hc])

            QS = CH // 4
            descs = []
            for q4, sg in enumerate((sg0, sg1, sg2, sg3)):
                descs.append(pltpu.async_copy(
                    xph.at[gidx.at[0, pl.ds(q4 * QS, QS)]],
                    rows.at[pl.ds(q4 * QS, QS)], sg))

            def scale_row(r, carry3):
                exs = plsc.load_gather(exb, [jnp.zeros((16,), jnp.int32) + r])
                for q in range(C // 16):
                    rows[r, pl.ds(16 * q, 16)] = rows[r, pl.ds(16 * q, 16)] * exs
                return carry3

            for q4 in range(4):
                descs[q4].wait()
                lax.fori_loop(q4 * QS, (q4 + 1) * QS, scale_row, 0)
            didx = dstb.at[0]
            pltpu.sync_copy(rows, accm.at[didx], add=True)
            pltpu.sync_copy(exb, accd.at[didx], add=True)

            @pl.when(czon)
            def _():
                pltpu.sync_copy(oneb.at[0], accc.at[didx], add=True)
                pltpu.sync_copy(zcb.at[0], az0.at[didx], add=True)
                pltpu.sync_copy(zcb.at[1], az1.at[didx], add=True)
                pltpu.sync_copy(zcb.at[2], az2.at[didx], add=True)
                pltpu.sync_copy(zcb.at[3], az3.at[didx], add=True)

            return carry2

        lax.fori_loop(0, NCHUNK, chunk, 0)
        plsc.subcore_barrier()
        pltpu.sync_copy(accm.at[pl.ds(r0, RPT)], xagg.at[pl.ds(hN + r0, RPT)])
        pltpu.sync_copy(accd.at[pl.ds(r0, RPT)], fb.at[0])
        pltpu.sync_copy(fb.at[0], den_out.at[pl.ds(hN + r0, RPT)])

        @pl.when(s == NS - 1)
        def _():
            pltpu.sync_copy(accm.at[pl.ds(NS * RPT, TAIL)],
                            xagg.at[pl.ds(hN + NS * RPT, TAIL)])
            pltpu.sync_copy(accd.at[pl.ds(NS * RPT, TAIL)],
                            fb.at[0, pl.ds(0, TAIL)])
            pltpu.sync_copy(fb.at[0, pl.ds(0, TAIL)],
                            den_out.at[pl.ds(hN + NS * RPT, TAIL)])

        @pl.when(czon)
        def _():
            for q, acc in enumerate((accc, az0, az1, az2, az3)):
                pltpu.sync_copy(acc.at[pl.ds(r0, RPT)], fb.at[0])
                pltpu.sync_copy(fb.at[0], cz_out.at[pl.ds(q * N + r0, RPT)])

            @pl.when(s == NS - 1)
            def _():
                for q, acc in enumerate((accc, az0, az1, az2, az3)):
                    pltpu.sync_copy(acc.at[pl.ds(NS * RPT, TAIL)],
                                    fb.at[0, pl.ds(0, TAIL)])
                    pltpu.sync_copy(fb.at[0, pl.ds(0, TAIL)],
                                    cz_out.at[pl.ds(q * N + NS * RPT, TAIL)])

        plsc.subcore_barrier()
        return carry

    lax.fori_loop(0, 2, head_pass, 0)


@functools.cache
def _build_edge_pass():
    return functools.partial(
        pl.kernel,
        mesh=plsc.VectorSubcoreMesh(core_axis_name="c", subcore_axis_name="s"),
        compiler_params=pltpu.CompilerParams(needs_layout_passes=False),
        out_type=(
            jax.ShapeDtypeStruct((H * N, C), jnp.float32),
            jax.ShapeDtypeStruct((H * N,), jnp.float32),
            jax.ShapeDtypeStruct((5 * N,), jnp.float32),
        ),
        scratch_types=[
            pltpu.VMEM_SHARED((N, C), jnp.float32),
            pltpu.VMEM_SHARED((N,), jnp.float32),
            pltpu.VMEM_SHARED((N,), jnp.float32),
            pltpu.VMEM_SHARED((N,), jnp.float32),
            pltpu.VMEM_SHARED((N,), jnp.float32),
            pltpu.VMEM_SHARED((N,), jnp.float32),
            pltpu.VMEM_SHARED((N,), jnp.float32),
            pltpu.VMEM((N,), jnp.float32),
            pltpu.VMEM((N,), jnp.float32),
            pltpu.VMEM((1, CH), jnp.int32),
            pltpu.VMEM((1, CH), jnp.int32),
            pltpu.VMEM((1, CH), jnp.int32),
            pltpu.VMEM((CH * H,), jnp.float32),
            pltpu.VMEM((CH, C), jnp.float32),
            pltpu.VMEM((CH,), jnp.float32),
            pltpu.VMEM((1, CH), jnp.float32),
            pltpu.VMEM((H, CH), jnp.float32),
            pltpu.VMEM((1, RPT), jnp.float32),
            pltpu.VMEM((1, RPT), jnp.float32),
            pltpu.SemaphoreType.DMA,
            pltpu.SemaphoreType.DMA,
            pltpu.SemaphoreType.DMA,
            pltpu.SemaphoreType.DMA,
        ],
    )(_edge_pass_body)


def _combine_body(xp_ref, xg0, xg1, xg2, xg3, den_ref, cz_ref, as_ref, ad_ref,
                  bias_ref, out_ref, stats_ref):
    i = pl.program_id(0)
    denom4 = den_ref[...]
    cnt = cz_ref[...][:, 0:1]
    zsum4 = cz_ref[...][:, 1:1 + H]
    zloop4 = zsum4 / jnp.maximum(cnt, 1.0)
    sa = as_ref[...] + ad_ref[...] + zloop4
    alpha = jnp.where(sa >= 0.0, sa, 0.2 * sa)
    exloop4 = jnp.exp(alpha)
    jj = lax.broadcasted_iota(jnp.int32, (H, HC), 1) // C
    hh = lax.broadcasted_iota(jnp.int32, (H, HC), 0)
    expand = (jj == hh).astype(jnp.float32)
    xagg = jnp.concatenate([xg0[...], xg1[...], xg2[...], xg3[...]], axis=1)
    num = xagg + xp_ref[...] * jnp.dot(exloop4, expand,
                                       preferred_element_type=jnp.float32)
    den = jnp.dot(denom4 + exloop4, expand,
                  preferred_element_type=jnp.float32) + 1e-16
    out = num / den + bias_ref[...]
    out_ref[...] = out

    @pl.when(i == 0)
    def _():
        stats_ref[...] = jnp.zeros_like(stats_ref)

    sums = jnp.sum(out, axis=0, keepdims=True)
    sq = jnp.sum(out * out, axis=0, keepdims=True)
    stats_ref[...] += jnp.concatenate([sums, sq], axis=0)


_combine = pl.pallas_call(
    _combine_body,
    grid=(N // RB,),
    in_specs=[
        pl.BlockSpec((RB, HC), lambda i: (i, 0)),
        pl.BlockSpec((RB, C), lambda i: (0 * (N // RB) + i, 0)),
        pl.BlockSpec((RB, C), lambda i: (1 * (N // RB) + i, 0)),
        pl.BlockSpec((RB, C), lambda i: (2 * (N // RB) + i, 0)),
        pl.BlockSpec((RB, C), lambda i: (3 * (N // RB) + i, 0)),
        pl.BlockSpec((RB, H), lambda i: (i, 0)),
        pl.BlockSpec((RB, 5), lambda i: (i, 0)),
        pl.BlockSpec((RB, H), lambda i: (i, 0)),
        pl.BlockSpec((RB, H), lambda i: (i, 0)),
        pl.BlockSpec((1, HC), lambda i: (0, 0)),
    ],
    out_specs=[
        pl.BlockSpec((RB, HC), lambda i: (i, 0)),
        pl.BlockSpec((2, HC), lambda i: (0, 0)),
    ],
    out_shape=[
        jax.ShapeDtypeStruct((N, HC), jnp.float32),
        jax.ShapeDtypeStruct((2, HC), jnp.float32),
    ],
)


def _bn_body(o_ref, stats_ref, g_ref, b_ref, out_ref):
    mu = stats_ref[0:1, :] / N
    ex2 = stats_ref[1:2, :] / N
    var = ex2 - mu * mu
    y = (o_ref[...] - mu) * lax.rsqrt(var + 1e-5) * g_ref[...] + b_ref[...]
    out_ref[...] = jnp.where(y > 0.0, y, jnp.exp(y) - 1.0)


_bn_elu = pl.pallas_call(
    _bn_body,
    grid=(N // RB,),
    in_specs=[
        pl.BlockSpec((RB, HC), lambda i: (i, 0)),
        pl.BlockSpec((2, HC), lambda i: (0, 0)),
        pl.BlockSpec((1, HC), lambda i: (0, 0)),
        pl.BlockSpec((1, HC), lambda i: (0, 0)),
    ],
    out_specs=pl.BlockSpec((RB, HC), lambda i: (i, 0)),
    out_shape=jax.ShapeDtypeStruct((N, HC), jnp.float32),
)


def kernel(x, edge_index, edge_attr, W, att_src, att_dst, W_edge, att_edge,
           bias, gamma, beta):
    src = edge_index[0]
    dst = edge_index[1]
    xp, xph, asrc4, adst4 = _proj(x, W, att_src.T, att_dst.T)
    z0 = _edgeproj(edge_attr, W_edge, att_edge)
    xagg, den, cz = _build_edge_pass()(xph, src, dst, z0.reshape(-1),
                                       asrc4.T.reshape(-1), adst4.T.reshape(-1))
    out_pre, stats = _combine(xp, xagg, xagg, xagg, xagg,
                              den.reshape(H, N).T, cz.reshape(5, N).T,
                              asrc4, adst4, bias.reshape(1, HC))
    return _bn_elu(out_pre, stats, gamma.reshape(1, HC), beta.reshape(1, HC))


# speedup vs baseline: 15.2516x; 1.2930x over previous
import functools

import jax
import jax.numpy as jnp
from jax import lax
from jax.experimental import pallas as pl
from jax.experimental.pallas import tpu as pltpu
from jax.experimental.pallas import tpu_sc as plsc

N = 10000
E = 160000
IN = 256
H = 4
C = 128
HC = H * C
ED = 16

RB = 400
EB = 2000
NS = 16
EPT = E // NS
CH = 80
NCHUNK = EPT // CH
RPT = 624
SUP = 2000


def _proj_body(x_ref, w_ref, as_ref, ad_ref, xp_ref, xph_ref, a4s_ref, a4d_ref):
    h = pl.program_id(1)
    xpb = jnp.dot(x_ref[...], w_ref[...], preferred_element_type=jnp.float32)
    xp_ref[...] = xpb
    xph_ref[...] = xpb
    a_s = jnp.dot(xpb, as_ref[...], preferred_element_type=jnp.float32)
    a_d = jnp.dot(xpb, ad_ref[...], preferred_element_type=jnp.float32)
    col = lax.broadcasted_iota(jnp.int32, (1, H), 1)

    @pl.when(h == 0)
    def _():
        a4s_ref[...] = jnp.zeros_like(a4s_ref)
        a4d_ref[...] = jnp.zeros_like(a4d_ref)

    a4s_ref[...] = jnp.where(col == h, a_s, a4s_ref[...])
    a4d_ref[...] = jnp.where(col == h, a_d, a4d_ref[...])


_proj = pl.pallas_call(
    _proj_body,
    grid=(N // RB, H),
    in_specs=[
        pl.BlockSpec((RB, IN), lambda i, h: (i, 0)),
        pl.BlockSpec((IN, C), lambda i, h: (0, h)),
        pl.BlockSpec((C, H), lambda i, h: (0, 0)),
        pl.BlockSpec((C, H), lambda i, h: (0, 0)),
    ],
    out_specs=[
        pl.BlockSpec((RB, C), lambda i, h: (i, h)),
        pl.BlockSpec((RB, C), lambda i, h: (h * (N // RB) + i, 0)),
        pl.BlockSpec((RB, H), lambda i, h: (i, 0)),
        pl.BlockSpec((RB, H), lambda i, h: (i, 0)),
    ],
    out_shape=[
        jax.ShapeDtypeStruct((N, HC), jnp.float32),
        jax.ShapeDtypeStruct((H * N, C), jnp.float32),
        jax.ShapeDtypeStruct((N, H), jnp.float32),
        jax.ShapeDtypeStruct((N, H), jnp.float32),
    ],
)


def _edgeproj_body(ea_ref, we_ref, ae_ref, z0_ref):
    ve = jnp.sum(we_ref[...].reshape(ED, H, C) * ae_ref[...][None], axis=2)
    z0_ref[...] = jnp.dot(ea_ref[...], ve, preferred_element_type=jnp.float32)


_edgeproj = pl.pallas_call(
    _edgeproj_body,
    grid=(E // EB,),
    in_specs=[
        pl.BlockSpec((EB, ED), lambda i: (i, 0)),
        pl.BlockSpec((ED, HC), lambda i: (0, 0)),
        pl.BlockSpec((H, C), lambda i: (0, 0)),
    ],
    out_specs=pl.BlockSpec((EB, H), lambda i: (i, 0)),
    out_shape=jax.ShapeDtypeStruct((E, H), jnp.float32),
)


def _edge_pass_body(xph, srcv, dstv, z0, asrc4, adst4, xagg, den_out, cz_out,
                    accm, accd, accc, az0, az1, az2, az3,
                    asrc_t, adst_t, srcs, dsts, dstb, gidx, zb,
                    rows, exb, oneb, zcb, zrb, fb,
                    sg0, sg1, sg2, sg3):
    c = lax.axis_index("c")
    s = lax.axis_index("s")
    r0 = s * RPT
    TAIL = N - NS * RPT
    iota16 = jnp.arange(16, dtype=jnp.int32)
    zeros16f = jnp.zeros((16,), jnp.float32)

    for g in range(CH // 16):
        oneb[0, pl.ds(16 * g, 16)] = zeros16f + 1.0
    for q in range(RPT // 16):
        zrb[0, pl.ds(16 * q, 16)] = zeros16f

    def head_pass(p, carry):
        h = 2 * c + p
        hN = h * N
        czon = jnp.logical_and(c == 0, p == 0)
        pltpu.sync_copy(asrc4.at[pl.ds(hN, N)], asrc_t)
        pltpu.sync_copy(adst4.at[pl.ds(hN, N)], adst_t)
        for i in range(CH):
            for q in range(C // 16):
                rows[i, pl.ds(16 * q, 16)] = zeros16f
        for k in range(7):
            pltpu.sync_copy(rows, accm.at[pl.ds(r0 + k * CH, CH)])
        pltpu.sync_copy(rows.at[pl.ds(0, RPT - 7 * CH)],
                        accm.at[pl.ds(r0 + 7 * CH, RPT - 7 * CH)])
        for acc in (accd, accc, az0, az1, az2, az3):
            pltpu.sync_copy(zrb.at[0], acc.at[pl.ds(r0, RPT)])

        @pl.when(s == NS - 1)
        def _():
            pltpu.sync_copy(rows.at[pl.ds(0, TAIL)],
                            accm.at[pl.ds(NS * RPT, TAIL)])
            for acc in (accd, accc, az0, az1, az2, az3):
                pltpu.sync_copy(zrb.at[0, pl.ds(0, TAIL)],
                                acc.at[pl.ds(NS * RPT, TAIL)])

        plsc.subcore_barrier()
        hsplat = jnp.full((16,), h, jnp.int32)

        def superchunk(sp, carry2):
            e0 = s * EPT + sp * SUP
            pltpu.sync_copy(srcv.at[pl.ds(e0, SUP)], srcs.at[0])
            pltpu.sync_copy(dstv.at[pl.ds(e0, SUP)], dsts.at[0])
            pltpu.sync_copy(z0.at[pl.ds(e0 * H, SUP * H)], zb)

            def chunk(j2, carry3):
                off = j2 * CH
                for g in range(CH // 16):
                    li = iota16 + off + 16 * g
                    sv = srcs[0, pl.ds(off + 16 * g, 16)]
                    dv = dsts[0, pl.ds(off + 16 * g, 16)]
                    dstb[0, pl.ds(16 * g, 16)] = dv
                    gidx[0, pl.ds(16 * g, 16)] = sv + hN
                    av = plsc.load_gather(asrc_t, [sv])
                    bv = plsc.load_gather(adst_t, [dv])
                    zv = plsc.load_gather(zb, [li * H + hsplat])
                    sa = av + bv + zv
                    alpha = jnp.where(sa >= 0.0, sa, 0.2 * sa)
                    exb[pl.ds(16 * g, 16)] = jnp.exp(alpha)

                @pl.when(czon)
                def _():
                    for g in range(CH // 16):
                        li = iota16 + off + 16 * g
                        for hc in range(H):
                            zcb[hc, pl.ds(16 * g, 16)] = plsc.load_gather(
                                zb, [li * H + hc])

                QS = CH // 4
                descs = []
                for q4, sg in enumerate((sg0, sg1, sg2, sg3)):
                    descs.append(pltpu.async_copy(
                        xph.at[gidx.at[0, pl.ds(q4 * QS, QS)]],
                        rows.at[pl.ds(q4 * QS, QS)], sg))

                def scale_row(r, carry4):
                    exs = plsc.load_gather(exb,
                                           [jnp.zeros((16,), jnp.int32) + r])
                    for q in range(C // 16):
                        rows[r, pl.ds(16 * q, 16)] = (
                            rows[r, pl.ds(16 * q, 16)] * exs)
                    return carry4

                for q4 in range(4):
                    descs[q4].wait()
                    lax.fori_loop(q4 * QS, (q4 + 1) * QS, scale_row, 0)
                didx = dstb.at[0]
                pltpu.sync_copy(rows, accm.at[didx], add=True)
                pltpu.sync_copy(exb, accd.at[didx], add=True)

                @pl.when(czon)
                def _():
                    pltpu.sync_copy(oneb.at[0], accc.at[didx], add=True)
                    pltpu.sync_copy(zcb.at[0], az0.at[didx], add=True)
                    pltpu.sync_copy(zcb.at[1], az1.at[didx], add=True)
                    pltpu.sync_copy(zcb.at[2], az2.at[didx], add=True)
                    pltpu.sync_copy(zcb.at[3], az3.at[didx], add=True)

                return carry3

            lax.fori_loop(0, SUP // CH, chunk, 0)
            return carry2

        lax.fori_loop(0, EPT // SUP, superchunk, 0)
        plsc.subcore_barrier()
        pltpu.sync_copy(accm.at[pl.ds(r0, RPT)], xagg.at[pl.ds(hN + r0, RPT)])
        pltpu.sync_copy(accd.at[pl.ds(r0, RPT)], fb.at[0])
        pltpu.sync_copy(fb.at[0], den_out.at[pl.ds(hN + r0, RPT)])

        @pl.when(s == NS - 1)
        def _():
            pltpu.sync_copy(accm.at[pl.ds(NS * RPT, TAIL)],
                            xagg.at[pl.ds(hN + NS * RPT, TAIL)])
            pltpu.sync_copy(accd.at[pl.ds(NS * RPT, TAIL)],
                            fb.at[0, pl.ds(0, TAIL)])
            pltpu.sync_copy(fb.at[0, pl.ds(0, TAIL)],
                            den_out.at[pl.ds(hN + NS * RPT, TAIL)])

        @pl.when(czon)
        def _():
            for q, acc in enumerate((accc, az0, az1, az2, az3)):
                pltpu.sync_copy(acc.at[pl.ds(r0, RPT)], fb.at[0])
                pltpu.sync_copy(fb.at[0], cz_out.at[pl.ds(q * N + r0, RPT)])

            @pl.when(s == NS - 1)
            def _():
                for q, acc in enumerate((accc, az0, az1, az2, az3)):
                    pltpu.sync_copy(acc.at[pl.ds(NS * RPT, TAIL)],
                                    fb.at[0, pl.ds(0, TAIL)])
                    pltpu.sync_copy(fb.at[0, pl.ds(0, TAIL)],
                                    cz_out.at[pl.ds(q * N + NS * RPT, TAIL)])

        plsc.subcore_barrier()
        return carry

    lax.fori_loop(0, 2, head_pass, 0)


@functools.cache
def _build_edge_pass():
    return functools.partial(
        pl.kernel,
        mesh=plsc.VectorSubcoreMesh(core_axis_name="c", subcore_axis_name="s"),
        compiler_params=pltpu.CompilerParams(needs_layout_passes=False),
        out_type=(
            jax.ShapeDtypeStruct((H * N, C), jnp.float32),
            jax.ShapeDtypeStruct((H * N,), jnp.float32),
            jax.ShapeDtypeStruct((5 * N,), jnp.float32),
        ),
        scratch_types=[
            pltpu.VMEM_SHARED((N, C), jnp.float32),
            pltpu.VMEM_SHARED((N,), jnp.float32),
            pltpu.VMEM_SHARED((N,), jnp.float32),
            pltpu.VMEM_SHARED((N,), jnp.float32),
            pltpu.VMEM_SHARED((N,), jnp.float32),
            pltpu.VMEM_SHARED((N,), jnp.float32),
            pltpu.VMEM_SHARED((N,), jnp.float32),
            pltpu.VMEM((N,), jnp.float32),
            pltpu.VMEM((N,), jnp.float32),
            pltpu.VMEM((1, SUP), jnp.int32),
            pltpu.VMEM((1, SUP), jnp.int32),
            pltpu.VMEM((1, CH), jnp.int32),
            pltpu.VMEM((1, CH), jnp.int32),
            pltpu.VMEM((SUP * H,), jnp.float32),
            pltpu.VMEM((CH, C), jnp.float32),
            pltpu.VMEM((CH,), jnp.float32),
            pltpu.VMEM((1, CH), jnp.float32),
            pltpu.VMEM((H, CH), jnp.float32),
            pltpu.VMEM((1, RPT), jnp.float32),
            pltpu.VMEM((1, RPT), jnp.float32),
            pltpu.SemaphoreType.DMA,
            pltpu.SemaphoreType.DMA,
            pltpu.SemaphoreType.DMA,
            pltpu.SemaphoreType.DMA,
        ],
    )(_edge_pass_body)


def _combine_body(xp_ref, xg0, xg1, xg2, xg3, den_ref, cz_ref, as_ref, ad_ref,
                  bias_ref, out_ref, stats_ref):
    i = pl.program_id(0)
    denom4 = den_ref[...]
    cnt = cz_ref[...][:, 0:1]
    zsum4 = cz_ref[...][:, 1:1 + H]
    zloop4 = zsum4 / jnp.maximum(cnt, 1.0)
    sa = as_ref[...] + ad_ref[...] + zloop4
    alpha = jnp.where(sa >= 0.0, sa, 0.2 * sa)
    exloop4 = jnp.exp(alpha)
    jj = lax.broadcasted_iota(jnp.int32, (H, HC), 1) // C
    hh = lax.broadcasted_iota(jnp.int32, (H, HC), 0)
    expand = (jj == hh).astype(jnp.float32)
    xagg = jnp.concatenate([xg0[...], xg1[...], xg2[...], xg3[...]], axis=1)
    num = xagg + xp_ref[...] * jnp.dot(exloop4, expand,
                                       preferred_element_type=jnp.float32)
    den = jnp.dot(denom4 + exloop4, expand,
                  preferred_element_type=jnp.float32) + 1e-16
    out = num / den + bias_ref[...]
    out_ref[...] = out

    @pl.when(i == 0)
    def _():
        stats_ref[...] = jnp.zeros_like(stats_ref)

    sums = jnp.sum(out, axis=0, keepdims=True)
    sq = jnp.sum(out * out, axis=0, keepdims=True)
    stats_ref[...] += jnp.concatenate([sums, sq], axis=0)


_combine = pl.pallas_call(
    _combine_body,
    grid=(N // RB,),
    in_specs=[
        pl.BlockSpec((RB, HC), lambda i: (i, 0)),
        pl.BlockSpec((RB, C), lambda i: (0 * (N // RB) + i, 0)),
        pl.BlockSpec((RB, C), lambda i: (1 * (N // RB) + i, 0)),
        pl.BlockSpec((RB, C), lambda i: (2 * (N // RB) + i, 0)),
        pl.BlockSpec((RB, C), lambda i: (3 * (N // RB) + i, 0)),
        pl.BlockSpec((RB, H), lambda i: (i, 0)),
        pl.BlockSpec((RB, 5), lambda i: (i, 0)),
        pl.BlockSpec((RB, H), lambda i: (i, 0)),
        pl.BlockSpec((RB, H), lambda i: (i, 0)),
        pl.BlockSpec((1, HC), lambda i: (0, 0)),
    ],
    out_specs=[
        pl.BlockSpec((RB, HC), lambda i: (i, 0)),
        pl.BlockSpec((2, HC), lambda i: (0, 0)),
    ],
    out_shape=[
        jax.ShapeDtypeStruct((N, HC), jnp.float32),
        jax.ShapeDtypeStruct((2, HC), jnp.float32),
    ],
)


def _bn_body(o_ref, stats_ref, g_ref, b_ref, out_ref):
    mu = stats_ref[0:1, :] / N
    ex2 = stats_ref[1:2, :] / N
    var = ex2 - mu * mu
    y = (o_ref[...] - mu) * lax.rsqrt(var + 1e-5) * g_ref[...] + b_ref[...]
    out_ref[...] = jnp.where(y > 0.0, y, jnp.exp(y) - 1.0)


_bn_elu = pl.pallas_call(
    _bn_body,
    grid=(N // RB,),
    in_specs=[
        pl.BlockSpec((RB, HC), lambda i: (i, 0)),
        pl.BlockSpec((2, HC), lambda i: (0, 0)),
        pl.BlockSpec((1, HC), lambda i: (0, 0)),
        pl.BlockSpec((1, HC), lambda i: (0, 0)),
    ],
    out_specs=pl.BlockSpec((RB, HC), lambda i: (i, 0)),
    out_shape=jax.ShapeDtypeStruct((N, HC), jnp.float32),
)


def kernel(x, edge_index, edge_attr, W, att_src, att_dst, W_edge, att_edge,
           bias, gamma, beta):
    src = edge_index[0]
    dst = edge_index[1]
    xp, xph, asrc4, adst4 = _proj(x, W, att_src.T, att_dst.T)
    z0 = _edgeproj(edge_attr, W_edge, att_edge)
    xagg, den, cz = _build_edge_pass()(xph, src, dst, z0.reshape(-1),
                                       asrc4.T.reshape(-1), adst4.T.reshape(-1))
    out_pre, stats = _combine(xp, xagg, xagg, xagg, xagg,
                              den.reshape(H, N).T, cz.reshape(5, N).T,
                              asrc4, adst4, bias.reshape(1, HC))
    return _bn_elu(out_pre, stats, gamma.reshape(1, HC), beta.reshape(1, HC))


# speedup vs baseline: 18.3962x; 1.2062x over previous
import functools

import jax
import jax.numpy as jnp
from jax import lax
from jax.experimental import pallas as pl
from jax.experimental.pallas import tpu as pltpu
from jax.experimental.pallas import tpu_sc as plsc

N = 10000
E = 160000
IN = 256
H = 4
C = 128
HC = H * C
ED = 16

RB = 400
EB = 2000
NS = 16
EPT = E // NS
CH = 80
NCHUNK = EPT // CH
RPT = 624
SUP = 2000


def _proj_body(x_ref, w_ref, as_ref, ad_ref, xp_ref, xph_ref, a4s_ref, a4d_ref):
    h = pl.program_id(1)
    xpb = jnp.dot(x_ref[...], w_ref[...], preferred_element_type=jnp.float32)
    xp_ref[...] = xpb
    xph_ref[...] = xpb
    a_s = jnp.dot(xpb, as_ref[...], preferred_element_type=jnp.float32)
    a_d = jnp.dot(xpb, ad_ref[...], preferred_element_type=jnp.float32)
    col = lax.broadcasted_iota(jnp.int32, (1, H), 1)

    @pl.when(h == 0)
    def _():
        a4s_ref[...] = jnp.zeros_like(a4s_ref)
        a4d_ref[...] = jnp.zeros_like(a4d_ref)

    a4s_ref[...] = jnp.where(col == h, a_s, a4s_ref[...])
    a4d_ref[...] = jnp.where(col == h, a_d, a4d_ref[...])


_proj = pl.pallas_call(
    _proj_body,
    grid=(N // RB, H),
    in_specs=[
        pl.BlockSpec((RB, IN), lambda i, h: (i, 0)),
        pl.BlockSpec((IN, C), lambda i, h: (0, h)),
        pl.BlockSpec((C, H), lambda i, h: (0, 0)),
        pl.BlockSpec((C, H), lambda i, h: (0, 0)),
    ],
    out_specs=[
        pl.BlockSpec((RB, C), lambda i, h: (i, h)),
        pl.BlockSpec((RB, C), lambda i, h: (h * (N // RB) + i, 0)),
        pl.BlockSpec((RB, H), lambda i, h: (i, 0)),
        pl.BlockSpec((RB, H), lambda i, h: (i, 0)),
    ],
    out_shape=[
        jax.ShapeDtypeStruct((N, HC), jnp.float32),
        jax.ShapeDtypeStruct((H * N, C), jnp.float32),
        jax.ShapeDtypeStruct((N, H), jnp.float32),
        jax.ShapeDtypeStruct((N, H), jnp.float32),
    ],
)


def _edgeproj_body(ea_ref, we_ref, ae_ref, z0_ref):
    ve = jnp.sum(we_ref[...].reshape(ED, H, C) * ae_ref[...][None], axis=2)
    z0_ref[...] = jnp.dot(ea_ref[...], ve, preferred_element_type=jnp.float32)


_edgeproj = pl.pallas_call(
    _edgeproj_body,
    grid=(E // EB,),
    in_specs=[
        pl.BlockSpec((EB, ED), lambda i: (i, 0)),
        pl.BlockSpec((ED, HC), lambda i: (0, 0)),
        pl.BlockSpec((H, C), lambda i: (0, 0)),
    ],
    out_specs=pl.BlockSpec((EB, H), lambda i: (i, 0)),
    out_shape=jax.ShapeDtypeStruct((E, H), jnp.float32),
)


def _edge_pass_body(xph, srcv, dstv, z0, asrc4, adst4, xagg, den_out, cz_out,
                    accm, accd, accc, az0, az1, az2, az3,
                    asrc_t, adst_t, srcs, dsts, dstb, gidx, zb,
                    rows, exb, oneb, zcb, zrb, fb,
                    sg0, sg1, sg2, sg3):
    c = lax.axis_index("c")
    s = lax.axis_index("s")
    r0 = s * RPT
    TAIL = N - NS * RPT
    iota16 = jnp.arange(16, dtype=jnp.int32)
    zeros16f = jnp.zeros((16,), jnp.float32)

    for g in range(CH // 16):
        oneb[0, pl.ds(16 * g, 16)] = zeros16f + 1.0
    for q in range(RPT // 16):
        zrb[0, pl.ds(16 * q, 16)] = zeros16f

    def head_pass(p, carry):
        h = 2 * c + p
        hN = h * N
        czon = jnp.logical_and(c == 0, p == 0)
        pltpu.sync_copy(asrc4.at[pl.ds(hN, N)], asrc_t)
        pltpu.sync_copy(adst4.at[pl.ds(hN, N)], adst_t)
        for i in range(CH):
            for q in range(C // 16):
                rows[i, pl.ds(16 * q, 16)] = zeros16f
        for k in range(7):
            pltpu.sync_copy(rows, accm.at[pl.ds(r0 + k * CH, CH)])
        pltpu.sync_copy(rows.at[pl.ds(0, RPT - 7 * CH)],
                        accm.at[pl.ds(r0 + 7 * CH, RPT - 7 * CH)])
        for acc in (accd, accc, az0, az1, az2, az3):
            pltpu.sync_copy(zrb.at[0], acc.at[pl.ds(r0, RPT)])

        @pl.when(s == NS - 1)
        def _():
            pltpu.sync_copy(rows.at[pl.ds(0, TAIL)],
                            accm.at[pl.ds(NS * RPT, TAIL)])
            for acc in (accd, accc, az0, az1, az2, az3):
                pltpu.sync_copy(zrb.at[0, pl.ds(0, TAIL)],
                                acc.at[pl.ds(NS * RPT, TAIL)])

        plsc.subcore_barrier()
        hsplat = jnp.full((16,), h, jnp.int32)

        def superchunk(sp, carry2):
            e0 = s * EPT + sp * SUP
            pltpu.sync_copy(srcv.at[pl.ds(e0, SUP)], srcs.at[0])
            pltpu.sync_copy(dstv.at[pl.ds(e0, SUP)], dsts.at[0])
            pltpu.sync_copy(z0.at[pl.ds(e0 * H, SUP * H)], zb)

            def chunk(j2, carry3):
                off = j2 * CH
                for g in range(CH // 16):
                    li = iota16 + off + 16 * g
                    sv = srcs[0, pl.ds(off + 16 * g, 16)]
                    dv = dsts[0, pl.ds(off + 16 * g, 16)]
                    dstb[0, pl.ds(16 * g, 16)] = dv
                    gidx[0, pl.ds(16 * g, 16)] = sv + hN
                    av = plsc.load_gather(asrc_t, [sv])
                    bv = plsc.load_gather(adst_t, [dv])
                    zv = plsc.load_gather(zb, [li * H + hsplat])
                    sa = av + bv + zv
                    alpha = jnp.where(sa >= 0.0, sa, 0.2 * sa)
                    exb[pl.ds(16 * g, 16)] = jnp.exp(alpha)

                @pl.when(czon)
                def _():
                    for g in range(CH // 16):
                        li = iota16 + off + 16 * g
                        for hc in range(H):
                            zcb[hc, pl.ds(16 * g, 16)] = plsc.load_gather(
                                zb, [li * H + hc])

                QS = CH // 4
                descs = []
                for q4, sg in enumerate((sg0, sg1, sg2, sg3)):
                    descs.append(pltpu.async_copy(
                        xph.at[gidx.at[0, pl.ds(q4 * QS, QS)]],
                        rows.at[pl.ds(q4 * QS, QS)], sg))

                for q4 in range(4):
                    descs[q4].wait()

                    @functools.partial(plsc.parallel_loop, q4 * QS,
                                       (q4 + 1) * QS, unroll=4)
                    def _(r):
                        exs = plsc.load_gather(
                            exb, [jnp.zeros((16,), jnp.int32) + r])
                        for q in range(C // 16):
                            rows[r, pl.ds(16 * q, 16)] = (
                                rows[r, pl.ds(16 * q, 16)] * exs)
                didx = dstb.at[0]
                pltpu.sync_copy(rows, accm.at[didx], add=True)
                pltpu.sync_copy(exb, accd.at[didx], add=True)

                @pl.when(czon)
                def _():
                    pltpu.sync_copy(oneb.at[0], accc.at[didx], add=True)
                    pltpu.sync_copy(zcb.at[0], az0.at[didx], add=True)
                    pltpu.sync_copy(zcb.at[1], az1.at[didx], add=True)
                    pltpu.sync_copy(zcb.at[2], az2.at[didx], add=True)
                    pltpu.sync_copy(zcb.at[3], az3.at[didx], add=True)

                return carry3

            lax.fori_loop(0, SUP // CH, chunk, 0)
            return carry2

        lax.fori_loop(0, EPT // SUP, superchunk, 0)
        plsc.subcore_barrier()
        pltpu.sync_copy(accm.at[pl.ds(r0, RPT)], xagg.at[pl.ds(hN + r0, RPT)])
        pltpu.sync_copy(accd.at[pl.ds(r0, RPT)], fb.at[0])
        pltpu.sync_copy(fb.at[0], den_out.at[pl.ds(hN + r0, RPT)])

        @pl.when(s == NS - 1)
        def _():
            pltpu.sync_copy(accm.at[pl.ds(NS * RPT, TAIL)],
                            xagg.at[pl.ds(hN + NS * RPT, TAIL)])
            pltpu.sync_copy(accd.at[pl.ds(NS * RPT, TAIL)],
                            fb.at[0, pl.ds(0, TAIL)])
            pltpu.sync_copy(fb.at[0, pl.ds(0, TAIL)],
                            den_out.at[pl.ds(hN + NS * RPT, TAIL)])

        @pl.when(czon)
        def _():
            for q, acc in enumerate((accc, az0, az1, az2, az3)):
                pltpu.sync_copy(acc.at[pl.ds(r0, RPT)], fb.at[0])
                pltpu.sync_copy(fb.at[0], cz_out.at[pl.ds(q * N + r0, RPT)])

            @pl.when(s == NS - 1)
            def _():
                for q, acc in enumerate((accc, az0, az1, az2, az3)):
                    pltpu.sync_copy(acc.at[pl.ds(NS * RPT, TAIL)],
                                    fb.at[0, pl.ds(0, TAIL)])
                    pltpu.sync_copy(fb.at[0, pl.ds(0, TAIL)],
                                    cz_out.at[pl.ds(q * N + NS * RPT, TAIL)])

        plsc.subcore_barrier()
        return carry

    lax.fori_loop(0, 2, head_pass, 0)


@functools.cache
def _build_edge_pass():
    return functools.partial(
        pl.kernel,
        mesh=plsc.VectorSubcoreMesh(core_axis_name="c", subcore_axis_name="s"),
        compiler_params=pltpu.CompilerParams(needs_layout_passes=False),
        out_type=(
            jax.ShapeDtypeStruct((H * N, C), jnp.float32),
            jax.ShapeDtypeStruct((H * N,), jnp.float32),
            jax.ShapeDtypeStruct((5 * N,), jnp.float32),
        ),
        scratch_types=[
            pltpu.VMEM_SHARED((N, C), jnp.float32),
            pltpu.VMEM_SHARED((N,), jnp.float32),
            pltpu.VMEM_SHARED((N,), jnp.float32),
            pltpu.VMEM_SHARED((N,), jnp.float32),
            pltpu.VMEM_SHARED((N,), jnp.float32),
            pltpu.VMEM_SHARED((N,), jnp.float32),
            pltpu.VMEM_SHARED((N,), jnp.float32),
            pltpu.VMEM((N,), jnp.float32),
            pltpu.VMEM((N,), jnp.float32),
            pltpu.VMEM((1, SUP), jnp.int32),
            pltpu.VMEM((1, SUP), jnp.int32),
            pltpu.VMEM((1, CH), jnp.int32),
            pltpu.VMEM((1, CH), jnp.int32),
            pltpu.VMEM((SUP * H,), jnp.float32),
            pltpu.VMEM((CH, C), jnp.float32),
            pltpu.VMEM((CH,), jnp.float32),
            pltpu.VMEM((1, CH), jnp.float32),
            pltpu.VMEM((H, CH), jnp.float32),
            pltpu.VMEM((1, RPT), jnp.float32),
            pltpu.VMEM((1, RPT), jnp.float32),
            pltpu.SemaphoreType.DMA,
            pltpu.SemaphoreType.DMA,
            pltpu.SemaphoreType.DMA,
            pltpu.SemaphoreType.DMA,
        ],
    )(_edge_pass_body)


def _combine_body(xp_ref, xg0, xg1, xg2, xg3, den_ref, cz_ref, as_ref, ad_ref,
                  bias_ref, out_ref, stats_ref):
    i = pl.program_id(0)
    denom4 = den_ref[...]
    cnt = cz_ref[...][:, 0:1]
    zsum4 = cz_ref[...][:, 1:1 + H]
    zloop4 = zsum4 / jnp.maximum(cnt, 1.0)
    sa = as_ref[...] + ad_ref[...] + zloop4
    alpha = jnp.where(sa >= 0.0, sa, 0.2 * sa)
    exloop4 = jnp.exp(alpha)
    jj = lax.broadcasted_iota(jnp.int32, (H, HC), 1) // C
    hh = lax.broadcasted_iota(jnp.int32, (H, HC), 0)
    expand = (jj == hh).astype(jnp.float32)
    xagg = jnp.concatenate([xg0[...], xg1[...], xg2[...], xg3[...]], axis=1)
    num = xagg + xp_ref[...] * jnp.dot(exloop4, expand,
                                       preferred_element_type=jnp.float32)
    den = jnp.dot(denom4 + exloop4, expand,
                  preferred_element_type=jnp.float32) + 1e-16
    out = num / den + bias_ref[...]
    out_ref[...] = out

    @pl.when(i == 0)
    def _():
        stats_ref[...] = jnp.zeros_like(stats_ref)

    sums = jnp.sum(out, axis=0, keepdims=True)
    sq = jnp.sum(out * out, axis=0, keepdims=True)
    stats_ref[...] += jnp.concatenate([sums, sq], axis=0)


_combine = pl.pallas_call(
    _combine_body,
    grid=(N // RB,),
    in_specs=[
        pl.BlockSpec((RB, HC), lambda i: (i, 0)),
        pl.BlockSpec((RB, C), lambda i: (0 * (N // RB) + i, 0)),
        pl.BlockSpec((RB, C), lambda i: (1 * (N // RB) + i, 0)),
        pl.BlockSpec((RB, C), lambda i: (2 * (N // RB) + i, 0)),
        pl.BlockSpec((RB, C), lambda i: (3 * (N // RB) + i, 0)),
        pl.BlockSpec((RB, H), lambda i: (i, 0)),
        pl.BlockSpec((RB, 5), lambda i: (i, 0)),
        pl.BlockSpec((RB, H), lambda i: (i, 0)),
        pl.BlockSpec((RB, H), lambda i: (i, 0)),
        pl.BlockSpec((1, HC), lambda i: (0, 0)),
    ],
    out_specs=[
        pl.BlockSpec((RB, HC), lambda i: (i, 0)),
        pl.BlockSpec((2, HC), lambda i: (0, 0)),
    ],
    out_shape=[
        jax.ShapeDtypeStruct((N, HC), jnp.float32),
        jax.ShapeDtypeStruct((2, HC), jnp.float32),
    ],
)


def _bn_body(o_ref, stats_ref, g_ref, b_ref, out_ref):
    mu = stats_ref[0:1, :] / N
    ex2 = stats_ref[1:2, :] / N
    var = ex2 - mu * mu
    y = (o_ref[...] - mu) * lax.rsqrt(var + 1e-5) * g_ref[...] + b_ref[...]
    out_ref[...] = jnp.where(y > 0.0, y, jnp.exp(y) - 1.0)


_bn_elu = pl.pallas_call(
    _bn_body,
    grid=(N // RB,),
    in_specs=[
        pl.BlockSpec((RB, HC), lambda i: (i, 0)),
        pl.BlockSpec((2, HC), lambda i: (0, 0)),
        pl.BlockSpec((1, HC), lambda i: (0, 0)),
        pl.BlockSpec((1, HC), lambda i: (0, 0)),
    ],
    out_specs=pl.BlockSpec((RB, HC), lambda i: (i, 0)),
    out_shape=jax.ShapeDtypeStruct((N, HC), jnp.float32),
)


def kernel(x, edge_index, edge_attr, W, att_src, att_dst, W_edge, att_edge,
           bias, gamma, beta):
    src = edge_index[0]
    dst = edge_index[1]
    xp, xph, asrc4, adst4 = _proj(x, W, att_src.T, att_dst.T)
    z0 = _edgeproj(edge_attr, W_edge, att_edge)
    xagg, den, cz = _build_edge_pass()(xph, src, dst, z0.reshape(-1),
                                       asrc4.T.reshape(-1), adst4.T.reshape(-1))
    out_pre, stats = _combine(xp, xagg, xagg, xagg, xagg,
                              den.reshape(H, N).T, cz.reshape(5, N).T,
                              asrc4, adst4, bias.reshape(1, HC))
    return _bn_elu(out_pre, stats, gamma.reshape(1, HC), beta.reshape(1, HC))
